# baseline (speedup 1.0000x reference)
import functools

import jax
import jax.numpy as jnp
from jax import lax
from jax.experimental import pallas as pl
from jax.experimental.pallas import tpu as pltpu
from jax.experimental.pallas import tpu_sc as plsc

_M_MESH = 50
_NSUB = 16


def _mlp(x, w1, b1, w2, b2, g=None, bt=None):
    h = jnp.dot(x, w1, preferred_element_type=jnp.float32) + b1
    h = jnp.maximum(h, 0.0)
    h = jnp.dot(h, w2, preferred_element_type=jnp.float32) + b2
    if g is not None:
        mu = jnp.mean(h, axis=-1, keepdims=True)
        var = jnp.mean((h - mu) * (h - mu), axis=-1, keepdims=True)
        h = (h - mu) * jax.lax.rsqrt(var + 1e-5) * g + bt
    return h


def _gather_onehot(idx_col, n_rows, n_cols):
    cols = jax.lax.broadcasted_iota(jnp.int32, (n_rows, n_cols), 1)
    return (cols == idx_col).astype(jnp.float32)


def _scatter_onehot_t(idx_row, n_rows, n_cols):
    rows = jax.lax.broadcasted_iota(jnp.int32, (n_rows, n_cols), 0)
    return (rows == idx_row).astype(jnp.float32)


def _mlp_of(pr, x, ln=True):
    if ln:
        return _mlp(x, pr[0][...], pr[1][...], pr[2][...], pr[3][...],
                    pr[4][...], pr[5][...])
    return _mlp(x, pr[0][...], pr[1][...], pr[2][...], pr[3][...])


def _body_pre(nbatch, *refs):
    feat_r = refs[0]
    enc = refs[1:7]
    gh_r = refs[7]
    for b in range(nbatch):
        gh_r[b] = _mlp_of(enc, feat_r[b])


def _sc_gather_call(table2, idx2, nbatch, e_pad, dim):
    chunk = e_pad // _NSUB
    mesh = plsc.VectorSubcoreMesh(core_axis_name="c", subcore_axis_name="s")

    @functools.partial(
        pl.kernel,
        mesh=mesh,
        out_type=jax.ShapeDtypeStruct((nbatch * e_pad, dim), jnp.float32),
        scratch_types=[
            pltpu.VMEM((chunk,), jnp.int32),
            pltpu.VMEM((chunk, dim), jnp.float32),
            pltpu.SemaphoreType.DMA,
        ],
    )
    def sc_gather(table_hbm, idx_hbm, out_hbm, idx_v, rows_v, sem):
        c = lax.axis_index("c")
        s = lax.axis_index("s")
        off = c * e_pad + s * chunk
        pltpu.sync_copy(idx_hbm.at[pl.ds(off, chunk)], idx_v)
        pltpu.async_copy(table_hbm.at[idx_v], rows_v, sem).wait()
        pltpu.sync_copy(rows_v, out_hbm.at[pl.ds(off, chunk)])

    return sc_gather(table2, idx2)


def _body_post(nbatch, nb, n_grid, m_mesh, e_g2m, e_mm, e_m2g, feat_dim, e_pad,
               *refs):
    (gath_r, gh_r, feat_r, g2ma_r, mma_r, m2ga_r,
     g2md_r, mms_r, mmdc_r, mmdr_r, m2gs_r, m2gd_r) = refs[:12]
    it = iter(refs[12:-1])
    out_r = refs[-1]

    def take6():
        return [next(it) for _ in range(6)]

    e_g2m_p = take6()
    g2m_e_p = take6()
    g2m_n_p = take6()
    e_mesh_p = take6()
    e_m2g_p = take6()
    proc_e_p = take6()
    proc_n_p = take6()
    m2g_e_p = take6()
    m2g_n_p = take6()
    dec_p = [next(it) for _ in range(4)]

    ge = _mlp_of(e_g2m_p, g2ma_r[...])
    me0 = _mlp_of(e_mesh_p, mma_r[...])
    de = _mlp_of(e_m2g_p, m2ga_r[...])
    s_g2m_t = _scatter_onehot_t(g2md_r[...], m_mesh, e_g2m)
    g_src = _gather_onehot(mms_r[...], e_mm, m_mesh)
    g_dst = _gather_onehot(mmdc_r[...], e_mm, m_mesh)
    s_mm_t = _scatter_onehot_t(mmdr_r[...], m_mesh, e_mm)
    g_m2g = _gather_onehot(m2gs_r[...], e_m2g, m_mesh)
    s_m2g_t = _scatter_onehot_t(m2gd_r[...], n_grid, e_m2g)

    latents, mes = [], []
    for b in range(nbatch):
        gathered = gath_r[b * e_pad:b * e_pad + e_g2m]
        msgs = _mlp_of(g2m_e_p, jnp.concatenate([gathered, ge], axis=1))
        agg = jnp.dot(s_g2m_t, msgs, preferred_element_type=jnp.float32)
        latents.append(_mlp_of(g2m_n_p, agg))
        mes.append(me0)

    for i in range(nb):
        pe = [proc_e_p[k][i] for k in range(6)]
        pn = [proc_n_p[k][i] for k in range(6)]
        for b in range(nbatch):
            latent, me = latents[b], mes[b]
            ein = jnp.concatenate(
                [jnp.dot(g_src, latent, preferred_element_type=jnp.float32),
                 jnp.dot(g_dst, latent, preferred_element_type=jnp.float32),
                 me], axis=1)
            me = me + _mlp(ein, *pe)
            agg = jnp.dot(s_mm_t, me, preferred_element_type=jnp.float32)
            latents[b] = latent + _mlp(jnp.concatenate([latent, agg], axis=1), *pn)
            mes[b] = me

    for b in range(nbatch):
        msgs = _mlp_of(m2g_e_p, jnp.concatenate(
            [jnp.dot(g_m2g, latents[b], preferred_element_type=jnp.float32), de],
            axis=1))
        aggn = jnp.dot(s_m2g_t, msgs, preferred_element_type=jnp.float32)
        node_h = _mlp_of(m2g_n_p, jnp.concatenate([aggn, gh_r[b]], axis=1))
        out = _mlp_of(dec_p, node_h, ln=False) + feat_r[b][:, :feat_dim]
        out_r[b] = out


def kernel(features, t, params, g2m_src, g2m_dst, g2m_attr,
           mm_src, mm_dst, mm_attr, m2g_src, m2g_dst, m2g_attr):
    del t
    b, n_grid, fin = features.shape
    m_mesh = _M_MESH
    e_g2m = g2m_src.shape[0]
    e_mm = mm_src.shape[0]
    e_m2g = m2g_src.shape[0]
    feat_dim = params['dec']['b2'].shape[0]
    nb = params['proc_e']['W1'].shape[0]
    dim = params['g2m_e']['W2'].shape[1]
    e_pad = ((e_g2m + _NSUB * 8 - 1) // (_NSUB * 8)) * (_NSUB * 8)

    def flat(d):
        return [d['W1'], d['b1'], d['W2'], d['b2'], d['g'], d['bt']]

    i32 = jnp.int32

    pre_args = [features] + flat(params['enc_node'])
    grid_h = pl.pallas_call(
        functools.partial(_body_pre, b),
        out_shape=jax.ShapeDtypeStruct((b, n_grid, dim), jnp.float32),
    )(*pre_args)

    src = g2m_src.astype(i32)
    src_pad = jnp.concatenate([src, jnp.zeros((e_pad - e_g2m,), i32)])
    idx2 = jnp.concatenate([src_pad + k * n_grid for k in range(b)])
    table2 = grid_h.reshape(b * n_grid, dim)
    gath = _sc_gather_call(table2, idx2, b, e_pad, dim)

    post_args = [gath, grid_h, features, g2m_attr, mm_attr, m2g_attr,
                 g2m_dst.astype(i32).reshape(1, e_g2m),
                 mm_src.astype(i32).reshape(e_mm, 1),
                 mm_dst.astype(i32).reshape(e_mm, 1),
                 mm_dst.astype(i32).reshape(1, e_mm),
                 m2g_src.astype(i32).reshape(e_m2g, 1),
                 m2g_dst.astype(i32).reshape(1, e_m2g)]
    post_args += flat(params['e_g2m'])
    post_args += flat(params['g2m_e'])
    post_args += flat(params['g2m_n'])
    post_args += flat(params['e_mesh'])
    post_args += flat(params['e_m2g'])
    post_args += flat(params['proc_e'])
    post_args += flat(params['proc_n'])
    post_args += flat(params['m2g_e'])
    post_args += flat(params['m2g_n'])
    post_args += [params['dec']['W1'], params['dec']['b1'],
                  params['dec']['W2'], params['dec']['b2']]
    return pl.pallas_call(
        functools.partial(_body_post, b, nb, n_grid, m_mesh, e_g2m, e_mm,
                          e_m2g, feat_dim, e_pad),
        out_shape=jax.ShapeDtypeStruct((b, n_grid, feat_dim), jnp.float32),
    )(*post_args)
